# Initial kernel scaffold; baseline (speedup 1.0000x reference)
import jax
import jax.numpy as jnp
from jax.experimental import pallas as pl


def kernel(x, x_0, Wlin, b, gamma, beta_bn):
    raise NotImplementedError("write your pallas kernel here")


# speedup vs baseline: 17.4361x; 17.4361x over previous
import jax
import jax.numpy as jnp
from jax.experimental import pallas as pl

_ALPHA = 0.1
_BETA = 0.5
_K = 9
_EPS = 1e-5
_INF = float("inf")


def _phase_a(x_ref, x0_ref, w_ref, b_ref, pre_ref):
    A = x_ref[0]
    A0 = x0_ref[0]
    C, N = A.shape
    G = jax.lax.dot_general(A, A, (((0,), (0,)), ((), ())),
                            preferred_element_type=jnp.float32)
    sq = jnp.sum(A * A, axis=0, keepdims=True)
    score = sq - 2.0 * G
    col = jax.lax.broadcasted_iota(jnp.int32, (N, N), 1)
    mask = jnp.zeros((N, N), jnp.float32)
    for _ in range(_K):
        rmin = jnp.min(score, axis=1, keepdims=True)
        hit = score == rmin
        idx = jnp.min(jnp.where(hit, col, N), axis=1, keepdims=True)
        sel = col == idx
        mask = jnp.where(sel, 1.0, mask)
        score = jnp.where(sel, _INF, score)
    nm = jax.lax.dot_general(A, mask * (1.0 / _K), (((1,), (1,)), ((), ())),
                             preferred_element_type=jnp.float32)
    h = (1.0 - _ALPHA) * nm + _ALPHA * A0
    lin = jnp.dot(w_ref[...], h, preferred_element_type=jnp.float32) + b_ref[...]
    pre_ref[0] = (1.0 - _BETA) * h + _BETA * lin


def _phase_b(pre_ref, r_ref, g_ref, bb_ref, out_ref):
    B, C, N = pre_ref.shape
    s1 = pre_ref[0]
    for i in range(1, B):
        s1 = s1 + pre_ref[i]
    mean = jnp.sum(s1, axis=1, keepdims=True) * (1.0 / (B * N))
    s2 = jnp.zeros((C, 1), jnp.float32)
    for i in range(B):
        d = pre_ref[i] - mean
        s2 = s2 + jnp.sum(d * d, axis=1, keepdims=True)
    var = s2 * (1.0 / (B * N))
    scale = g_ref[...] * jax.lax.rsqrt(var + _EPS)
    shift = bb_ref[...] - mean * scale
    for i in range(B):
        out_ref[i] = jnp.maximum(pre_ref[i] * scale + shift + r_ref[i], 0.0)


def kernel(x, x_0, Wlin, b, gamma, beta_bn):
    B, C, H, W = x.shape
    N = H * W
    x3 = x.reshape(B, C, N)
    x03 = x_0.reshape(B, C, N)
    b2 = b.reshape(C, 1)
    g2 = gamma.reshape(C, 1)
    bb2 = beta_bn.reshape(C, 1)

    pre = pl.pallas_call(
        _phase_a,
        grid=(B,),
        in_specs=[
            pl.BlockSpec((1, C, N), lambda i: (i, 0, 0)),
            pl.BlockSpec((1, C, N), lambda i: (i, 0, 0)),
            pl.BlockSpec((C, C), lambda i: (0, 0)),
            pl.BlockSpec((C, 1), lambda i: (0, 0)),
        ],
        out_specs=pl.BlockSpec((1, C, N), lambda i: (i, 0, 0)),
        out_shape=jax.ShapeDtypeStruct((B, C, N), jnp.float32),
    )(x3, x03, Wlin, b2)

    out = pl.pallas_call(
        _phase_b,
        out_shape=jax.ShapeDtypeStruct((B, C, N), jnp.float32),
    )(pre, x3, g2, bb2)
    return out.reshape(B, C, H, W)


# speedup vs baseline: 31.1338x; 1.7856x over previous
import jax
import jax.numpy as jnp
from jax.experimental import pallas as pl

_ALPHA = 0.1
_BETA = 0.5
_K = 9
_EPS = 1e-5
_INF = float("inf")


def _phase_a(x_ref, x0_ref, w_ref, b_ref, pre_ref):
    A = x_ref[0]
    A0 = x0_ref[0]
    C, N = A.shape
    G = jax.lax.dot_general(A, A, (((0,), (0,)), ((), ())),
                            preferred_element_type=jnp.float32)
    sq = jnp.sum(A * A, axis=0, keepdims=True)
    score = sq - 2.0 * G
    for _ in range(_K):
        rmin = jnp.min(score, axis=1, keepdims=True)
        score = jnp.where(score == rmin, _INF, score)
    mask = (score == _INF).astype(jnp.float32)
    cnt = jnp.sum(mask, axis=1, keepdims=True)
    nm = jax.lax.dot_general(A, mask * (1.0 / cnt), (((1,), (1,)), ((), ())),
                             preferred_element_type=jnp.float32)
    h = (1.0 - _ALPHA) * nm + _ALPHA * A0
    lin = jnp.dot(w_ref[...], h, preferred_element_type=jnp.float32) + b_ref[...]
    pre_ref[0] = (1.0 - _BETA) * h + _BETA * lin


def _phase_b(pre_ref, r_ref, g_ref, bb_ref, out_ref):
    B, C, N = pre_ref.shape
    s1 = pre_ref[0]
    for i in range(1, B):
        s1 = s1 + pre_ref[i]
    mean = jnp.sum(s1, axis=1, keepdims=True) * (1.0 / (B * N))
    s2 = jnp.zeros((C, 1), jnp.float32)
    for i in range(B):
        d = pre_ref[i] - mean
        s2 = s2 + jnp.sum(d * d, axis=1, keepdims=True)
    var = s2 * (1.0 / (B * N))
    scale = g_ref[...] * jax.lax.rsqrt(var + _EPS)
    shift = bb_ref[...] - mean * scale
    for i in range(B):
        out_ref[i] = jnp.maximum(pre_ref[i] * scale + shift + r_ref[i], 0.0)


def kernel(x, x_0, Wlin, b, gamma, beta_bn):
    B, C, H, W = x.shape
    N = H * W
    x3 = x.reshape(B, C, N)
    x03 = x_0.reshape(B, C, N)
    b2 = b.reshape(C, 1)
    g2 = gamma.reshape(C, 1)
    bb2 = beta_bn.reshape(C, 1)

    pre = pl.pallas_call(
        _phase_a,
        grid=(B,),
        in_specs=[
            pl.BlockSpec((1, C, N), lambda i: (i, 0, 0)),
            pl.BlockSpec((1, C, N), lambda i: (i, 0, 0)),
            pl.BlockSpec((C, C), lambda i: (0, 0)),
            pl.BlockSpec((C, 1), lambda i: (0, 0)),
        ],
        out_specs=pl.BlockSpec((1, C, N), lambda i: (i, 0, 0)),
        out_shape=jax.ShapeDtypeStruct((B, C, N), jnp.float32),
    )(x3, x03, Wlin, b2)

    out = pl.pallas_call(
        _phase_b,
        out_shape=jax.ShapeDtypeStruct((B, C, N), jnp.float32),
    )(pre, x3, g2, bb2)
    return out.reshape(B, C, H, W)


# speedup vs baseline: 31.1538x; 1.0006x over previous
import jax
import jax.numpy as jnp
from jax.experimental import pallas as pl
from jax.experimental.pallas import tpu as pltpu

_ALPHA = 0.1
_BETA = 0.5
_K = 9
_EPS = 1e-5
_INF = float("inf")


def _fused(x_ref, x0_ref, w_ref, b_ref, g_ref, bb_ref, out_ref, pre_ref):
    B, C, N = x_ref.shape
    i = pl.program_id(0)
    A = x_ref[i]
    G = jax.lax.dot_general(A, A, (((0,), (0,)), ((), ())),
                            preferred_element_type=jnp.float32)
    sq = jnp.sum(A * A, axis=0, keepdims=True)
    score = sq - 2.0 * G
    for _ in range(_K):
        rmin = jnp.min(score, axis=1, keepdims=True)
        score = jnp.where(score == rmin, _INF, score)
    mask = (score == _INF).astype(jnp.float32)
    cnt = jnp.sum(mask, axis=1, keepdims=True)
    nm = jax.lax.dot_general(A, mask * (1.0 / cnt), (((1,), (1,)), ((), ())),
                             preferred_element_type=jnp.float32)
    h = (1.0 - _ALPHA) * nm + _ALPHA * x0_ref[i]
    lin = jnp.dot(w_ref[...], h, preferred_element_type=jnp.float32) + b_ref[...]
    pre_ref[i] = (1.0 - _BETA) * h + _BETA * lin

    @pl.when(i == B - 1)
    def _bn():
        s1 = pre_ref[0]
        for j in range(1, B):
            s1 = s1 + pre_ref[j]
        mean = jnp.sum(s1, axis=1, keepdims=True) * (1.0 / (B * N))
        s2 = jnp.zeros((C, 1), jnp.float32)
        for j in range(B):
            d = pre_ref[j] - mean
            s2 = s2 + jnp.sum(d * d, axis=1, keepdims=True)
        var = s2 * (1.0 / (B * N))
        scale = g_ref[...] * jax.lax.rsqrt(var + _EPS)
        shift = bb_ref[...] - mean * scale
        for j in range(B):
            out_ref[j] = jnp.maximum(pre_ref[j] * scale + shift + x_ref[j], 0.0)


def kernel(x, x_0, Wlin, b, gamma, beta_bn):
    B, C, H, W = x.shape
    N = H * W
    x3 = x.reshape(B, C, N)
    x03 = x_0.reshape(B, C, N)
    b2 = b.reshape(C, 1)
    g2 = gamma.reshape(C, 1)
    bb2 = beta_bn.reshape(C, 1)

    full3 = pl.BlockSpec((B, C, N), lambda i: (0, 0, 0))
    col = pl.BlockSpec((C, 1), lambda i: (0, 0))
    out = pl.pallas_call(
        _fused,
        grid=(B,),
        in_specs=[
            full3,
            full3,
            pl.BlockSpec((C, C), lambda i: (0, 0)),
            col, col, col,
        ],
        out_specs=full3,
        out_shape=jax.ShapeDtypeStruct((B, C, N), jnp.float32),
        scratch_shapes=[pltpu.VMEM((B, C, N), jnp.float32)],
    )(x3, x03, Wlin, b2, g2, bb2)
    return out.reshape(B, C, H, W)
